# baseline (speedup 1.0000x reference)
import functools

import jax
import jax.numpy as jnp
from jax import lax
from jax.experimental import pallas as pl
from jax.experimental.pallas import tpu as pltpu
from jax.experimental.pallas import tpu_sc as plsc

_N = 10000
_K = 16
_ALPHA = 0.5

_QB = 200
_NQ = _N // _QB
_NB = 400
_ND = _N // _NB

_GW = 128
_GH = 128

_NW = 32
_TR = 128
_TPW = 40
_BPAD = _NW * _TPW * _TR


def _knn_body(xq_ref, xall_ref, sqq_ref, sqa_ref, nbr_ref):
    i = pl.program_id(0)
    xq = xq_ref[...]
    g = lax.dot_general(xq, xall_ref[...], (((1,), (1,)), ((), ())),
                        preferred_element_type=jnp.float32)
    d = sqq_ref[...] + sqa_ref[...] - 2.0 * g
    rows = i * _QB + lax.broadcasted_iota(jnp.int32, (_QB, _N), 0)
    cols = lax.broadcasted_iota(jnp.int32, (_QB, _N), 1)
    inf = jnp.float32(jnp.inf)
    d = jnp.where(cols == rows, inf, d)
    ams = []
    for _ in range(_K):
        m = jnp.min(d, axis=1, keepdims=True)
        cand = jnp.where(d <= m, cols, _N)
        am = jnp.min(cand, axis=1, keepdims=True)
        ams.append(am)
        d = jnp.where(cols == am, inf, d)
    nbr_ref[...] = jnp.concatenate(ams, axis=1)


def _knn(x, sq):
    return pl.pallas_call(
        _knn_body,
        grid=(_NQ,),
        in_specs=[
            pl.BlockSpec((_QB, 128), lambda i: (i, 0)),
            pl.BlockSpec((_N, 128), lambda i: (0, 0)),
            pl.BlockSpec((_QB, 1), lambda i: (i, 0)),
            pl.BlockSpec((1, _N), lambda i: (0, 0)),
        ],
        out_specs=pl.BlockSpec((_QB, _K), lambda i: (i, 0)),
        out_shape=jax.ShapeDtypeStruct((_N, _K), jnp.int32),
        compiler_params=pltpu.CompilerParams(
            dimension_semantics=("arbitrary",)),
    )(x, x, sq[:, None], sq[None, :])


_NS = 5


def _sc_gather_wide(table, idx3, d_width):
    mesh = plsc.VectorSubcoreMesh(core_axis_name="c", subcore_axis_name="s")

    @functools.partial(
        pl.kernel,
        out_type=jax.ShapeDtypeStruct((_BPAD, d_width), jnp.float32),
        mesh=mesh,
        scratch_types=[
            pltpu.VMEM((_TPW, _TR), jnp.int32),
            pltpu.VMEM((_NS * _TR, d_width), jnp.float32),
        ] + [pltpu.SemaphoreType.DMA] * (2 * _NS),
    )
    def gath(table_hbm, idx_hbm, out_hbm, idx_v, rows_v, *sems):
        gsem = sems[:_NS]
        osem = sems[_NS:]
        c = lax.axis_index("c")
        s = lax.axis_index("s")
        w = s * 2 + c
        pltpu.sync_copy(idx_hbm.at[w], idx_v)
        base = w * (_TPW * _TR)

        for b in range(_NS):
            pltpu.async_copy(table_hbm.at[idx_v.at[b]],
                             rows_v.at[pl.ds(b * _TR, _TR)], gsem[b])

        ng = _TPW // _NS

        def outer(g, carry):
            for b in range(_NS):
                t = g * _NS + b
                slot = rows_v.at[pl.ds(b * _TR, _TR)]
                off = pl.multiple_of(base + t * _TR, _TR)
                dst = out_hbm.at[pl.ds(off, _TR)]
                pltpu.make_async_copy(table_hbm.at[idx_v.at[t]], slot,
                                      gsem[b]).wait()
                pltpu.async_copy(slot, dst, osem[b])
                pltpu.make_async_copy(slot, dst, osem[b]).wait()
                pltpu.async_copy(table_hbm.at[idx_v.at[t + _NS]], slot,
                                 gsem[b])
            return carry

        lax.fori_loop(0, ng - 1, outer, 0)
        for b in range(_NS):
            t = (ng - 1) * _NS + b
            slot = rows_v.at[pl.ds(b * _TR, _TR)]
            off = pl.multiple_of(base + t * _TR, _TR)
            dst = out_hbm.at[pl.ds(off, _TR)]
            pltpu.make_async_copy(table_hbm.at[idx_v.at[t]], slot,
                                  gsem[b]).wait()
            pltpu.async_copy(slot, dst, osem[b])
        for b in range(_NS):
            t = (ng - 1) * _NS + b
            slot = rows_v.at[pl.ds(b * _TR, _TR)]
            off = pl.multiple_of(base + t * _TR, _TR)
            pltpu.make_async_copy(slot, out_hbm.at[pl.ds(off, _TR)],
                                  osem[b]).wait()

    return gath(table, idx3)


def _gather_rows(table, idx3, d_width):
    out = _sc_gather_wide(table, idx3, d_width)
    return out[:_N * _K].reshape(_N, _K, d_width)


_NSH = 2
_PR = _TPW * _TR // 8


def _sc_gather_h(table, idx3):
    mesh = plsc.VectorSubcoreMesh(core_axis_name="c", subcore_axis_name="s")

    @functools.partial(
        pl.kernel,
        out_type=jax.ShapeDtypeStruct((_BPAD // 8, 128), jnp.float32),
        mesh=mesh,
        scratch_types=[
            pltpu.VMEM((_TPW, _TR), jnp.int32),
            pltpu.VMEM((_NSH * _TR, _GW), jnp.float32),
            pltpu.VMEM((_PR, 128), jnp.float32),
        ] + [pltpu.SemaphoreType.DMA] * _NSH,
    )
    def gath(table_hbm, idx_hbm, out_hbm, idx_v, rows_v, packed_v, *gsem):
        c = lax.axis_index("c")
        s = lax.axis_index("s")
        w = s * 2 + c
        pltpu.sync_copy(idx_hbm.at[w], idx_v)

        for b in range(_NSH):
            pltpu.async_copy(table_hbm.at[idx_v.at[b]],
                             rows_v.at[pl.ds(b * _TR, _TR)], gsem[b])

        def group(gi, carry):
            for b in range(_NSH):
                t = gi * _NSH + b
                slot = rows_v.at[pl.ds(b * _TR, _TR)]
                pltpu.make_async_copy(table_hbm.at[idx_v.at[t]], slot,
                                      gsem[b]).wait()

                def rep(q, carry2):
                    row = t * 16 + q
                    for j in range(8):
                        v = rows_v[b * _TR + 8 * q + j, pl.ds(0, 16)]
                        packed_v[row, pl.ds(16 * j, 16)] = v
                    return carry2

                lax.fori_loop(0, 16, rep, 0)
                nt = t + _NSH

                @pl.when(nt < _TPW)
                def _():
                    pltpu.async_copy(table_hbm.at[idx_v.at[nt]], slot,
                                     gsem[b])
            return carry

        lax.fori_loop(0, _TPW // _NSH, group, 0)
        pltpu.sync_copy(packed_v, out_hbm.at[pl.ds(w * _PR, _PR)])

    return gath(table, idx3)


def _unpack_hs(pg):
    return jnp.stack([pg[:, 16 * j:16 * j + 5] for j in range(8)],
                     axis=1).reshape(_NB * _K, 5)


def _padg(h):
    return jnp.concatenate(
        [h, jnp.zeros((h.shape[0], _GH - h.shape[1]), jnp.float32)], axis=1)


def _h0_body(x_ref, xg_ref, w1_ref, b1_ref, w2_ref, b2_ref, h0_ref):
    xd = x_ref[...]
    xdb = jnp.broadcast_to(xd[:, None, :], (_NB, _K, 128))
    u = jnp.concatenate([xdb, xg_ref[...] - xdb],
                        axis=2).reshape(_NB * _K, 256)
    z = jnp.maximum(_lin(u, w1_ref, b1_ref), 0.0)
    m = _lin(z, w2_ref, b2_ref).reshape(_NB, _K, 5)
    h = jnp.maximum(jnp.max(m, axis=1), 0.0)
    h0_ref[...] = _padg(h)


def _compute_h0(x, xg, w1, b1, w2, b2):
    return pl.pallas_call(
        _h0_body,
        grid=(_ND,),
        in_specs=[
            pl.BlockSpec((_NB, 128), lambda i: (i, 0)),
            pl.BlockSpec((_NB, _K, _GW), lambda i: (i, 0, 0)),
            pl.BlockSpec((256, 40), lambda i: (0, 0)),
            pl.BlockSpec((1, 40), lambda i: (0, 0)),
            pl.BlockSpec((40, 5), lambda i: (0, 0)),
            pl.BlockSpec((1, 5), lambda i: (0, 0)),
        ],
        out_specs=pl.BlockSpec((_NB, _GH), lambda i: (i, 0)),
        out_shape=jax.ShapeDtypeStruct((_N, _GH), jnp.float32),
        compiler_params=pltpu.CompilerParams(
            dimension_semantics=("arbitrary",)),
    )(x, xg, w1, b1, w2, b2)


def _lin(x, w_ref, b_ref):
    return (lax.dot_general(x, w_ref[...], (((1,), (0,)), ((), ())),
                            preferred_element_type=jnp.float32)
            + b_ref[...])


def _in_block(hs, hd, h, e, refs):
    we1, be1, we2, be2, wn1, bn1, wn2, bn2 = refs
    x1 = jnp.concatenate([hs, hd, e], axis=1)
    e_out = _lin(jnp.maximum(_lin(x1, we1, be1), 0.0), we2, be2)
    agg = e_out.reshape(_NB, _K, 8).sum(axis=1)
    xn = jnp.concatenate([h, agg], axis=1)
    dh = _lin(jnp.maximum(_lin(xn, wn1, bn1), 0.0), wn2, bn2)
    return e_out, _ALPHA * h + (1.0 - _ALPHA) * dh


def _enc_hc0_body(h_ref, hg_ref,
                  wee1, bee1, wee2, bee2,
                  we1, be1, we2, be2, wn1, bn1, wn2, bn2,
                  e0_ref, e1_ref, h1_ref):
    h = h_ref[:, :5]
    hs = _unpack_hs(hg_ref[...])
    hd = jnp.broadcast_to(h[:, None, :], (_NB, _K, 5)).reshape(_NB * _K, 5)
    x0 = jnp.concatenate([hs, hd], axis=1)
    e0 = jnp.maximum(
        _lin(jnp.maximum(_lin(x0, wee1, bee1), 0.0), wee2, bee2), 0.0)
    e1, h1 = _in_block(hs, hd, h, e0,
                       (we1, be1, we2, be2, wn1, bn1, wn2, bn2))
    e0_ref[...] = e0.reshape(_NB, _K, 8)
    e1_ref[...] = e1.reshape(_NB, _K, 8)
    h1_ref[...] = _padg(h1)


def _enc_hc0(h0p, hg0, wts):
    full = lambda a, b: pl.BlockSpec((a, b), lambda i: (0, 0))
    return pl.pallas_call(
        _enc_hc0_body,
        grid=(_ND,),
        in_specs=[
            pl.BlockSpec((_NB, _GH), lambda i: (i, 0)),
            pl.BlockSpec((_NB * _K // 8, 128), lambda i: (i, 0)),
        ] + [full(*w.shape) for w in wts],
        out_specs=[
            pl.BlockSpec((_NB, _K, 8), lambda i: (i, 0, 0)),
            pl.BlockSpec((_NB, _K, 8), lambda i: (i, 0, 0)),
            pl.BlockSpec((_NB, _GH), lambda i: (i, 0)),
        ],
        out_shape=[
            jax.ShapeDtypeStruct((_N, _K, 8), jnp.float32),
            jax.ShapeDtypeStruct((_N, _K, 8), jnp.float32),
            jax.ShapeDtypeStruct((_N, _GH), jnp.float32),
        ],
        compiler_params=pltpu.CompilerParams(
            dimension_semantics=("arbitrary",)),
    )(h0p, hg0, *wts)


def _hc_body(h_ref, hg_ref, e_ref,
             we1, be1, we2, be2, wn1, bn1, wn2, bn2,
             e_out_ref, h_out_ref):
    h = h_ref[:, :5]
    hs = _unpack_hs(hg_ref[...])
    hd = jnp.broadcast_to(h[:, None, :], (_NB, _K, 5)).reshape(_NB * _K, 5)
    e = e_ref[...].reshape(_NB * _K, 8)
    e_out, h_out = _in_block(hs, hd, h, e,
                             (we1, be1, we2, be2, wn1, bn1, wn2, bn2))
    e_out_ref[...] = e_out.reshape(_NB, _K, 8)
    h_out_ref[...] = _padg(h_out)


def _hc_round(hp, hg, e, wts):
    full = lambda a, b: pl.BlockSpec((a, b), lambda i: (0, 0))
    return pl.pallas_call(
        _hc_body,
        grid=(_ND,),
        in_specs=[
            pl.BlockSpec((_NB, _GH), lambda i: (i, 0)),
            pl.BlockSpec((_NB * _K // 8, 128), lambda i: (i, 0)),
            pl.BlockSpec((_NB, _K, 8), lambda i: (i, 0, 0)),
        ] + [full(*w.shape) for w in wts],
        out_specs=[
            pl.BlockSpec((_NB, _K, 8), lambda i: (i, 0, 0)),
            pl.BlockSpec((_NB, _GH), lambda i: (i, 0)),
        ],
        out_shape=[
            jax.ShapeDtypeStruct((_N, _K, 8), jnp.float32),
            jax.ShapeDtypeStruct((_N, _GH), jnp.float32),
        ],
        compiler_params=pltpu.CompilerParams(
            dimension_semantics=("arbitrary",)),
    )(hp, hg, e, *wts)


def _final_body(h_ref, hg_ref, e0_ref, e1_ref, e2_ref, e3_ref, *refs):
    (wb1, bb1, wb2, bb2, wb3, bb3, wb4, bb4,
     wx1, bx1, wx2, bx2, wx3, bx3, wx4, bx4,
     wpe1, bpe1, wpe2, bpe2, wpn1, bpn1, wpn2, bpn2,
     hout_ref, beta_ref, track_ref) = refs
    h = h_ref[:, :5]
    v = jnp.maximum(_lin(h, wb1, bb1), 0.0)
    v = jnp.maximum(_lin(v, wb2, bb2), 0.0)
    v = jnp.maximum(_lin(v, wb3, bb3), 0.0)
    beta_ref[...] = jax.nn.sigmoid(_lin(v, wb4, bb4)) + 1e-05
    v = jnp.maximum(_lin(h, wx1, bx1), 0.0)
    v = jnp.maximum(_lin(v, wx2, bx2), 0.0)
    v = jnp.maximum(_lin(v, wx3, bx3), 0.0)
    hout_ref[...] = _lin(v, wx4, bx4)
    hs = _unpack_hs(hg_ref[...])
    hd = jnp.broadcast_to(h[:, None, :], (_NB, _K, 5)).reshape(_NB * _K, 5)
    ecat = jnp.concatenate(
        [hs, hd,
         e0_ref[...].reshape(_NB * _K, 8), e1_ref[...].reshape(_NB * _K, 8),
         e2_ref[...].reshape(_NB * _K, 8), e3_ref[...].reshape(_NB * _K, 8)],
        axis=1)
    ep = _lin(jnp.maximum(_lin(ecat, wpe1, bpe1), 0.0), wpe2, bpe2)
    agg = ep.reshape(_NB, _K, 1).sum(axis=1)
    xn = jnp.concatenate([h, agg], axis=1)
    track_ref[...] = _lin(jnp.maximum(_lin(xn, wpn1, bpn1), 0.0), wpn2, bpn2)


def _final_heads(hp, hg, es, wts):
    full = lambda a, b: pl.BlockSpec((a, b), lambda i: (0, 0))
    return pl.pallas_call(
        _final_body,
        grid=(_ND,),
        in_specs=[
            pl.BlockSpec((_NB, _GH), lambda i: (i, 0)),
            pl.BlockSpec((_NB * _K // 8, 128), lambda i: (i, 0)),
        ] + [pl.BlockSpec((_NB, _K, 8), lambda i: (i, 0, 0))] * 4
          + [full(*w.shape) for w in wts],
        out_specs=[
            pl.BlockSpec((_NB, 2), lambda i: (i, 0)),
            pl.BlockSpec((_NB, 1), lambda i: (i, 0)),
            pl.BlockSpec((_NB, 1), lambda i: (i, 0)),
        ],
        out_shape=[
            jax.ShapeDtypeStruct((_N, 2), jnp.float32),
            jax.ShapeDtypeStruct((_N, 1), jnp.float32),
            jax.ShapeDtypeStruct((_N, 1), jnp.float32),
        ],
        compiler_params=pltpu.CompilerParams(
            dimension_semantics=("arbitrary",)),
    )(hp, hg, *es, *wts)


def _wb(params_list):
    out = []
    for w, b in params_list:
        out.append(w)
        out.append(b[None, :])
    return tuple(out)


def kernel(x, params):
    sq = jnp.sum(x * x, axis=1)
    w1, b1 = params["node_encoder"][0]
    w2, b2 = params["node_encoder"][1]

    nbr = _knn(x, sq)

    idx3 = jnp.concatenate(
        [nbr.reshape(-1),
         jnp.zeros((_BPAD - _N * _K,), jnp.int32)]).reshape(_NW, _TPW, _TR)

    xg = _gather_rows(x, idx3, _GW)
    h0p = _compute_h0(x, xg, w1, b1[None, :], w2, b2[None, :])

    hg = _sc_gather_h(h0p, idx3)
    wts0 = _wb(params["edge_encoder"]) + _wb(params["hc"][0]["edge"]) \
        + _wb(params["hc"][0]["node"])
    e0, e1, hp = _enc_hc0(h0p, hg, wts0)

    es = [e0, e1]
    for t in (1, 2):
        hg = _sc_gather_h(hp, idx3)
        wts = _wb(params["hc"][t]["edge"]) + _wb(params["hc"][t]["node"])
        e_next, hp = _hc_round(hp, hg, es[-1], wts)
        es.append(e_next)

    hg = _sc_gather_h(hp, idx3)
    wts_f = _wb(params["B"]) + _wb(params["X"]) \
        + _wb(params["P"]["edge"]) + _wb(params["P"]["node"])
    h_out, beta, track = _final_heads(hp, hg, es, wts_f)
    return (h_out, beta, track)


# speedup vs baseline: 1.0515x; 1.0515x over previous
import functools

import jax
import jax.numpy as jnp
from jax import lax
from jax.experimental import pallas as pl
from jax.experimental.pallas import tpu as pltpu
from jax.experimental.pallas import tpu_sc as plsc

_N = 10000
_K = 16
_ALPHA = 0.5

_QB = 200
_NQ = _N // _QB
_NB = 400
_ND = _N // _NB

_GW = 128
_GH = 128

_NW = 32
_TR = 128
_TPW = 40
_BPAD = _NW * _TPW * _TR


def _knn_body(xq_ref, xall_ref, sqq_ref, sqa_ref, nbr_ref):
    i = pl.program_id(0)
    xq = xq_ref[...]
    g = lax.dot_general(xq, xall_ref[...], (((1,), (1,)), ((), ())),
                        preferred_element_type=jnp.float32)
    d = sqq_ref[...] + sqa_ref[...] - 2.0 * g
    rows = i * _QB + lax.broadcasted_iota(jnp.int32, (_QB, _N), 0)
    cols = lax.broadcasted_iota(jnp.int32, (_QB, _N), 1)
    inf = jnp.float32(jnp.inf)
    d = jnp.where(cols == rows, inf, d)
    ams = []
    for _ in range(_K):
        am = jnp.argmin(d, axis=1)[:, None].astype(jnp.int32)
        ams.append(am)
        d = jnp.where(cols == am, inf, d)
    nbr_ref[...] = jnp.concatenate(ams, axis=1)


def _knn(x, sq):
    return pl.pallas_call(
        _knn_body,
        grid=(_NQ,),
        in_specs=[
            pl.BlockSpec((_QB, 128), lambda i: (i, 0)),
            pl.BlockSpec((_N, 128), lambda i: (0, 0)),
            pl.BlockSpec((_QB, 1), lambda i: (i, 0)),
            pl.BlockSpec((1, _N), lambda i: (0, 0)),
        ],
        out_specs=pl.BlockSpec((_QB, _K), lambda i: (i, 0)),
        out_shape=jax.ShapeDtypeStruct((_N, _K), jnp.int32),
        compiler_params=pltpu.CompilerParams(
            dimension_semantics=("arbitrary",)),
    )(x, x, sq[:, None], sq[None, :])


_NS = 5


def _sc_gather_wide(table, idx3, d_width):
    mesh = plsc.VectorSubcoreMesh(core_axis_name="c", subcore_axis_name="s")

    @functools.partial(
        pl.kernel,
        out_type=jax.ShapeDtypeStruct((_BPAD, d_width), jnp.float32),
        mesh=mesh,
        scratch_types=[
            pltpu.VMEM((_TPW, _TR), jnp.int32),
            pltpu.VMEM((_NS * _TR, d_width), jnp.float32),
        ] + [pltpu.SemaphoreType.DMA] * (2 * _NS),
    )
    def gath(table_hbm, idx_hbm, out_hbm, idx_v, rows_v, *sems):
        gsem = sems[:_NS]
        osem = sems[_NS:]
        c = lax.axis_index("c")
        s = lax.axis_index("s")
        w = s * 2 + c
        pltpu.sync_copy(idx_hbm.at[w], idx_v)
        base = w * (_TPW * _TR)

        for b in range(_NS):
            pltpu.async_copy(table_hbm.at[idx_v.at[b]],
                             rows_v.at[pl.ds(b * _TR, _TR)], gsem[b])

        ng = _TPW // _NS

        def outer(g, carry):
            for b in range(_NS):
                t = g * _NS + b
                slot = rows_v.at[pl.ds(b * _TR, _TR)]
                off = pl.multiple_of(base + t * _TR, _TR)
                dst = out_hbm.at[pl.ds(off, _TR)]
                pltpu.make_async_copy(table_hbm.at[idx_v.at[t]], slot,
                                      gsem[b]).wait()
                pltpu.async_copy(slot, dst, osem[b])
                pltpu.make_async_copy(slot, dst, osem[b]).wait()
                pltpu.async_copy(table_hbm.at[idx_v.at[t + _NS]], slot,
                                 gsem[b])
            return carry

        lax.fori_loop(0, ng - 1, outer, 0)
        for b in range(_NS):
            t = (ng - 1) * _NS + b
            slot = rows_v.at[pl.ds(b * _TR, _TR)]
            off = pl.multiple_of(base + t * _TR, _TR)
            dst = out_hbm.at[pl.ds(off, _TR)]
            pltpu.make_async_copy(table_hbm.at[idx_v.at[t]], slot,
                                  gsem[b]).wait()
            pltpu.async_copy(slot, dst, osem[b])
        for b in range(_NS):
            t = (ng - 1) * _NS + b
            slot = rows_v.at[pl.ds(b * _TR, _TR)]
            off = pl.multiple_of(base + t * _TR, _TR)
            pltpu.make_async_copy(slot, out_hbm.at[pl.ds(off, _TR)],
                                  osem[b]).wait()

    return gath(table, idx3)


def _gather_rows(table, idx3, d_width):
    out = _sc_gather_wide(table, idx3, d_width)
    return out[:_N * _K].reshape(_N, _K, d_width)


def _padg(h):
    return jnp.concatenate(
        [h, jnp.zeros((h.shape[0], _GH - h.shape[1]), jnp.float32)], axis=1)


def _h0_body(x_ref, xg_ref, w1_ref, b1_ref, w2_ref, b2_ref, h0_ref):
    xd = x_ref[...]
    xdb = jnp.broadcast_to(xd[:, None, :], (_NB, _K, 128))
    u = jnp.concatenate([xdb, xg_ref[...] - xdb],
                        axis=2).reshape(_NB * _K, 256)
    z = jnp.maximum(_lin(u, w1_ref, b1_ref), 0.0)
    m = _lin(z, w2_ref, b2_ref).reshape(_NB, _K, 5)
    h = jnp.maximum(jnp.max(m, axis=1), 0.0)
    h0_ref[...] = _padg(h)


def _compute_h0(x, xg, w1, b1, w2, b2):
    return pl.pallas_call(
        _h0_body,
        grid=(_ND,),
        in_specs=[
            pl.BlockSpec((_NB, 128), lambda i: (i, 0)),
            pl.BlockSpec((_NB, _K, _GW), lambda i: (i, 0, 0)),
            pl.BlockSpec((256, 40), lambda i: (0, 0)),
            pl.BlockSpec((1, 40), lambda i: (0, 0)),
            pl.BlockSpec((40, 5), lambda i: (0, 0)),
            pl.BlockSpec((1, 5), lambda i: (0, 0)),
        ],
        out_specs=pl.BlockSpec((_NB, _GH), lambda i: (i, 0)),
        out_shape=jax.ShapeDtypeStruct((_N, _GH), jnp.float32),
        compiler_params=pltpu.CompilerParams(
            dimension_semantics=("arbitrary",)),
    )(x, xg, w1, b1, w2, b2)


def _lin(x, w_ref, b_ref):
    return (lax.dot_general(x, w_ref[...], (((1,), (0,)), ((), ())),
                            preferred_element_type=jnp.float32)
            + b_ref[...])


def _in_block(hs, hd, h, e, refs):
    we1, be1, we2, be2, wn1, bn1, wn2, bn2 = refs
    x1 = jnp.concatenate([hs, hd, e], axis=1)
    e_out = _lin(jnp.maximum(_lin(x1, we1, be1), 0.0), we2, be2)
    agg = e_out.reshape(_NB, _K, 8).sum(axis=1)
    xn = jnp.concatenate([h, agg], axis=1)
    dh = _lin(jnp.maximum(_lin(xn, wn1, bn1), 0.0), wn2, bn2)
    return e_out, _ALPHA * h + (1.0 - _ALPHA) * dh


def _enc_hc0_body(h_ref, hg_ref,
                  wee1, bee1, wee2, bee2,
                  we1, be1, we2, be2, wn1, bn1, wn2, bn2,
                  e0_ref, e1_ref, h1_ref):
    h = h_ref[:, :5]
    hs = hg_ref[:, :, :5].reshape(_NB * _K, 5)
    hd = jnp.broadcast_to(h[:, None, :], (_NB, _K, 5)).reshape(_NB * _K, 5)
    x0 = jnp.concatenate([hs, hd], axis=1)
    e0 = jnp.maximum(
        _lin(jnp.maximum(_lin(x0, wee1, bee1), 0.0), wee2, bee2), 0.0)
    e1, h1 = _in_block(hs, hd, h, e0,
                       (we1, be1, we2, be2, wn1, bn1, wn2, bn2))
    e0_ref[...] = e0.reshape(_NB, _K, 8)
    e1_ref[...] = e1.reshape(_NB, _K, 8)
    h1_ref[...] = _padg(h1)


def _enc_hc0(h0p, hg0, wts):
    full = lambda a, b: pl.BlockSpec((a, b), lambda i: (0, 0))
    return pl.pallas_call(
        _enc_hc0_body,
        grid=(_ND,),
        in_specs=[
            pl.BlockSpec((_NB, _GH), lambda i: (i, 0)),
            pl.BlockSpec((_NB, _K, _GH), lambda i: (i, 0, 0)),
        ] + [full(*w.shape) for w in wts],
        out_specs=[
            pl.BlockSpec((_NB, _K, 8), lambda i: (i, 0, 0)),
            pl.BlockSpec((_NB, _K, 8), lambda i: (i, 0, 0)),
            pl.BlockSpec((_NB, _GH), lambda i: (i, 0)),
        ],
        out_shape=[
            jax.ShapeDtypeStruct((_N, _K, 8), jnp.float32),
            jax.ShapeDtypeStruct((_N, _K, 8), jnp.float32),
            jax.ShapeDtypeStruct((_N, _GH), jnp.float32),
        ],
        compiler_params=pltpu.CompilerParams(
            dimension_semantics=("arbitrary",)),
    )(h0p, hg0, *wts)


def _hc_body(h_ref, hg_ref, e_ref,
             we1, be1, we2, be2, wn1, bn1, wn2, bn2,
             e_out_ref, h_out_ref):
    h = h_ref[:, :5]
    hs = hg_ref[:, :, :5].reshape(_NB * _K, 5)
    hd = jnp.broadcast_to(h[:, None, :], (_NB, _K, 5)).reshape(_NB * _K, 5)
    e = e_ref[...].reshape(_NB * _K, 8)
    e_out, h_out = _in_block(hs, hd, h, e,
                             (we1, be1, we2, be2, wn1, bn1, wn2, bn2))
    e_out_ref[...] = e_out.reshape(_NB, _K, 8)
    h_out_ref[...] = _padg(h_out)


def _hc_round(hp, hg, e, wts):
    full = lambda a, b: pl.BlockSpec((a, b), lambda i: (0, 0))
    return pl.pallas_call(
        _hc_body,
        grid=(_ND,),
        in_specs=[
            pl.BlockSpec((_NB, _GH), lambda i: (i, 0)),
            pl.BlockSpec((_NB, _K, _GH), lambda i: (i, 0, 0)),
            pl.BlockSpec((_NB, _K, 8), lambda i: (i, 0, 0)),
        ] + [full(*w.shape) for w in wts],
        out_specs=[
            pl.BlockSpec((_NB, _K, 8), lambda i: (i, 0, 0)),
            pl.BlockSpec((_NB, _GH), lambda i: (i, 0)),
        ],
        out_shape=[
            jax.ShapeDtypeStruct((_N, _K, 8), jnp.float32),
            jax.ShapeDtypeStruct((_N, _GH), jnp.float32),
        ],
        compiler_params=pltpu.CompilerParams(
            dimension_semantics=("arbitrary",)),
    )(hp, hg, e, *wts)


def _final_body(h_ref, hg_ref, e0_ref, e1_ref, e2_ref, e3_ref, *refs):
    (wb1, bb1, wb2, bb2, wb3, bb3, wb4, bb4,
     wx1, bx1, wx2, bx2, wx3, bx3, wx4, bx4,
     wpe1, bpe1, wpe2, bpe2, wpn1, bpn1, wpn2, bpn2,
     hout_ref, beta_ref, track_ref) = refs
    h = h_ref[:, :5]
    v = jnp.maximum(_lin(h, wb1, bb1), 0.0)
    v = jnp.maximum(_lin(v, wb2, bb2), 0.0)
    v = jnp.maximum(_lin(v, wb3, bb3), 0.0)
    beta_ref[...] = jax.nn.sigmoid(_lin(v, wb4, bb4)) + 1e-05
    v = jnp.maximum(_lin(h, wx1, bx1), 0.0)
    v = jnp.maximum(_lin(v, wx2, bx2), 0.0)
    v = jnp.maximum(_lin(v, wx3, bx3), 0.0)
    hout_ref[...] = _lin(v, wx4, bx4)
    hs = hg_ref[:, :, :5].reshape(_NB * _K, 5)
    hd = jnp.broadcast_to(h[:, None, :], (_NB, _K, 5)).reshape(_NB * _K, 5)
    ecat = jnp.concatenate(
        [hs, hd,
         e0_ref[...].reshape(_NB * _K, 8), e1_ref[...].reshape(_NB * _K, 8),
         e2_ref[...].reshape(_NB * _K, 8), e3_ref[...].reshape(_NB * _K, 8)],
        axis=1)
    ep = _lin(jnp.maximum(_lin(ecat, wpe1, bpe1), 0.0), wpe2, bpe2)
    agg = ep.reshape(_NB, _K, 1).sum(axis=1)
    xn = jnp.concatenate([h, agg], axis=1)
    track_ref[...] = _lin(jnp.maximum(_lin(xn, wpn1, bpn1), 0.0), wpn2, bpn2)


def _final_heads(hp, hg, es, wts):
    full = lambda a, b: pl.BlockSpec((a, b), lambda i: (0, 0))
    return pl.pallas_call(
        _final_body,
        grid=(_ND,),
        in_specs=[
            pl.BlockSpec((_NB, _GH), lambda i: (i, 0)),
            pl.BlockSpec((_NB, _K, _GH), lambda i: (i, 0, 0)),
        ] + [pl.BlockSpec((_NB, _K, 8), lambda i: (i, 0, 0))] * 4
          + [full(*w.shape) for w in wts],
        out_specs=[
            pl.BlockSpec((_NB, 2), lambda i: (i, 0)),
            pl.BlockSpec((_NB, 1), lambda i: (i, 0)),
            pl.BlockSpec((_NB, 1), lambda i: (i, 0)),
        ],
        out_shape=[
            jax.ShapeDtypeStruct((_N, 2), jnp.float32),
            jax.ShapeDtypeStruct((_N, 1), jnp.float32),
            jax.ShapeDtypeStruct((_N, 1), jnp.float32),
        ],
        compiler_params=pltpu.CompilerParams(
            dimension_semantics=("arbitrary",)),
    )(hp, hg, *es, *wts)


def _wb(params_list):
    out = []
    for w, b in params_list:
        out.append(w)
        out.append(b[None, :])
    return tuple(out)


def kernel(x, params):
    sq = jnp.sum(x * x, axis=1)
    w1, b1 = params["node_encoder"][0]
    w2, b2 = params["node_encoder"][1]

    nbr = _knn(x, sq)

    idx3 = jnp.concatenate(
        [nbr.reshape(-1),
         jnp.zeros((_BPAD - _N * _K,), jnp.int32)]).reshape(_NW, _TPW, _TR)

    xg = _gather_rows(x, idx3, _GW)
    h0p = _compute_h0(x, xg, w1, b1[None, :], w2, b2[None, :])

    hg = _gather_rows(h0p, idx3, _GH)
    wts0 = _wb(params["edge_encoder"]) + _wb(params["hc"][0]["edge"]) \
        + _wb(params["hc"][0]["node"])
    e0, e1, hp = _enc_hc0(h0p, hg, wts0)

    es = [e0, e1]
    for t in (1, 2):
        hg = _gather_rows(hp, idx3, _GH)
        wts = _wb(params["hc"][t]["edge"]) + _wb(params["hc"][t]["node"])
        e_next, hp = _hc_round(hp, hg, es[-1], wts)
        es.append(e_next)

    hg = _gather_rows(hp, idx3, _GH)
    wts_f = _wb(params["B"]) + _wb(params["X"]) \
        + _wb(params["P"]["edge"]) + _wb(params["P"]["node"])
    h_out, beta, track = _final_heads(hp, hg, es, wts_f)
    return (h_out, beta, track)
